# baseline (speedup 1.0000x reference)
import functools

import jax
import jax.numpy as jnp
from jax import lax
from jax.experimental import pallas as pl
from jax.experimental.pallas import tpu as pltpu
from jax.experimental.pallas import tpu_sc as plsc

_B, _N, _H, _M = 1024, 200, 128, 200
_EPS = 1e-12
_T = _B * _N
_NC, _NS, _L = 2, 16, 16
_NW = _NC * _NS
_TW = _T // _NW
_C = 64
_NCHUNK = _TW // _C
_GPC = _C // _L
_HV = _H // _L
_GS = _M + 1


def _bcast_lane(vec, idx):
    return lax.gather(
        vec, idx[:, None],
        dimension_numbers=lax.GatherDimensionNumbers(
            offset_dims=(), collapsed_slice_dims=(0,), start_index_map=(0,)),
        slice_sizes=(1,),
        mode=lax.GatherScatterMode.PROMISE_IN_BOUNDS)


def _tc_stats_body(tbl_ref, o_ref):
    t = tbl_ref[:, :]
    g = lax.dot_general(t, t, (((1,), (1,)), ((), ())),
                        preferred_element_type=jnp.float32) * (1.0 / _H)
    m = jnp.mean(t, axis=1, keepdims=True)
    o_ref[:, : _M] = g
    o_ref[:, _M:] = m


def _sc_body(tbl2_h, gx_h, p_h, s_h, g_h, b_h, out_h,
             tbl_v, gx_v, g_v, b_v,
             pidx0, pidx1, sidx0, sidx1,
             rp0, rp1, rs0, rs1, out0, out1,
             semp0, semp1, sems0, sems1, semo0, semo1):
    pidx = [pidx0, pidx1]
    sidx = [sidx0, sidx1]
    rp = [rp0, rp1]
    rs = [rs0, rs1]
    out_v = [out0, out1]
    semp = [semp0, semp1]
    sems = [sems0, sems1]
    semo = [semo0, semo1]

    wid = lax.axis_index("s") * _NC + lax.axis_index("c")
    pltpu.sync_copy(tbl2_h, tbl_v)
    pltpu.sync_copy(gx_h, gx_v)
    pltpu.sync_copy(g_h, g_v)
    pltpu.sync_copy(b_h, b_v)
    base0 = wid * _TW
    lane = lax.iota(jnp.int32, _L)
    zf = jnp.zeros((_L,), jnp.float32)
    half = jnp.full((_L,), 0.5, jnp.float32)
    quarter = jnp.full((_L,), 0.25, jnp.float32)
    epsv = jnp.full((_L,), _EPS, jnp.float32)
    magic = jnp.full((_L,), 0x5F3759DF, jnp.int32)
    gs = [g_v[pl.ds(cv * _L, _L)] for cv in range(_HV)]
    bs = [b_v[pl.ds(cv * _L, _L)] for cv in range(_HV)]

    def stage_in(kk, b):
        base = base0 + kk * _C
        pltpu.sync_copy(p_h.at[pl.ds(base, _C)], pidx[b])
        pltpu.sync_copy(s_h.at[pl.ds(base, _C)], sidx[b])
        pltpu.async_copy(tbl2_h.at[pidx[b]], rp[b], semp[b])
        pltpu.async_copy(tbl2_h.at[sidx[b]], rs[b], sems[b])

    for b in range(2):
        stage_in(b, b)

    def chunk_pair(k2, carry):
        for b in range(2):
            kk = k2 * 2 + b
            base = base0 + kk * _C
            pltpu.make_async_copy(tbl2_h.at[pidx[b]], rp[b], semp[b]).wait()
            pltpu.make_async_copy(tbl2_h.at[sidx[b]], rs[b], sems[b]).wait()

            @pl.when(kk >= 2)
            def _wait_out():
                pltpu.make_async_copy(
                    out_v[b], out_h.at[pl.ds(0, _C * _H)], semo[b]).wait()

            rpb, rsb, ovb = rp[b], rs[b], out_v[b]

            for g in range(_GPC):
                tok0 = g * _L
                pv = pidx[b][pl.ds(tok0, _L)]
                sv = sidx[b][pl.ds(tok0, _L)]
                nv = lax.rem(lane + (base + tok0), _N)
                nr = nv * _GS
                pr = pv * _GS
                sr = sv * _GS
                m_n = plsc.load_gather(gx_v, [nr + _M])
                m_p = plsc.load_gather(gx_v, [pr + _M])
                m_s = plsc.load_gather(gx_v, [sr + _M])
                g_nn = plsc.load_gather(gx_v, [nr + nv])
                g_pp = plsc.load_gather(gx_v, [pr + pv])
                g_ss = plsc.load_gather(gx_v, [sr + sv])
                g_np = plsc.load_gather(gx_v, [nr + pv])
                g_ns = plsc.load_gather(gx_v, [nr + sv])
                g_ps = plsc.load_gather(gx_v, [pr + sv])
                mu = m_n + half * (m_p + m_s)
                e2 = (g_nn + quarter * (g_pp + g_ss)
                      + (g_np + g_ns) + half * g_ps)
                var = e2 - mu * mu + epsv
                yi = magic - (plsc.bitcast(var, jnp.int32) >> 1)
                y = plsc.bitcast(yi, jnp.float32)
                for _ in range(3):
                    y = y * (1.5 - 0.5 * var * y * y)

                @plsc.parallel_loop(0, _L, unroll=2)
                def _tok(t):
                    tsplat = jnp.zeros((_L,), jnp.int32) + t
                    mu_sp = _bcast_lane(mu, tsplat)
                    inv_sp = _bcast_lane(y, tsplat)
                    nb = lax.rem(base + tok0 + t, _N)
                    tt = tok0 + t
                    ob = tt * _H
                    for cv in range(_HV):
                        vn = tbl_v[nb, pl.ds(cv * _L, _L)]
                        vp = rpb[tt, pl.ds(cv * _L, _L)]
                        vs_ = rsb[tt, pl.ds(cv * _L, _L)]
                        e = vn + half * (vp + vs_)
                        ovb[pl.ds(ob + cv * _L, _L)] = (
                            (e - mu_sp) * inv_sp * gs[cv] + bs[cv])

            pltpu.async_copy(out_v[b], out_h.at[pl.ds(base * _H, _C * _H)],
                             semo[b])

            @pl.when(kk + 2 < _NCHUNK)
            def _prefetch():
                stage_in(kk + 2, b)
        return carry

    lax.fori_loop(0, _NCHUNK // 2, chunk_pair, 0)
    for b in range(2):
        pltpu.make_async_copy(
            out_v[b], out_h.at[pl.ds(0, _C * _H)], semo[b]).wait()


def kernel(top_vecs, tok_struct_vec, sent_struct_vec, table, gamma, beta):
    del top_vecs, tok_struct_vec
    p_idx = sent_struct_vec[:, :, 0].reshape(_T).astype(jnp.int32)
    s_idx = sent_struct_vec[:, :, 1].reshape(_T).astype(jnp.int32)
    gram = pl.pallas_call(
        _tc_stats_body,
        out_shape=jax.ShapeDtypeStruct((_M, _GS), jnp.float32),
    )(table)
    mesh = plsc.VectorSubcoreMesh(core_axis_name="c", subcore_axis_name="s")
    run = functools.partial(
        pl.kernel,
        mesh=mesh,
        compiler_params=pltpu.CompilerParams(needs_layout_passes=False),
        out_type=jax.ShapeDtypeStruct((_T * _H,), jnp.float32),
        scratch_types=[
            pltpu.VMEM((_M, _H), jnp.float32),
            pltpu.VMEM((_M * _GS,), jnp.float32),
            pltpu.VMEM((_H,), jnp.float32),
            pltpu.VMEM((_H,), jnp.float32),
            pltpu.VMEM((_C,), jnp.int32),
            pltpu.VMEM((_C,), jnp.int32),
            pltpu.VMEM((_C,), jnp.int32),
            pltpu.VMEM((_C,), jnp.int32),
            pltpu.VMEM((_C, _H), jnp.float32),
            pltpu.VMEM((_C, _H), jnp.float32),
            pltpu.VMEM((_C, _H), jnp.float32),
            pltpu.VMEM((_C, _H), jnp.float32),
            pltpu.VMEM((_C * _H,), jnp.float32),
            pltpu.VMEM((_C * _H,), jnp.float32),
            pltpu.SemaphoreType.DMA,
            pltpu.SemaphoreType.DMA,
            pltpu.SemaphoreType.DMA,
            pltpu.SemaphoreType.DMA,
            pltpu.SemaphoreType.DMA,
            pltpu.SemaphoreType.DMA,
        ],
    )(_sc_body)
    out = run(table, gram.reshape(_M * _GS), p_idx, s_idx, gamma, beta)
    return out.reshape(_B, _N, _H)


# speedup vs baseline: 1.9067x; 1.9067x over previous
import functools

import jax
import jax.numpy as jnp
from jax import lax
from jax.experimental import pallas as pl
from jax.experimental.pallas import tpu as pltpu
from jax.experimental.pallas import tpu_sc as plsc

_B, _N, _H, _M = 1024, 200, 128, 200
_EPS = 1e-12
_T = _B * _N
_NC, _NS, _L = 2, 16, 16
_NW = _NC * _NS
_TW = _T // _NW
_C = 128
_NCHUNK = _TW // _C
_G = _C // _L
_HV = _H // _L
_CP = _H // 2
_PS = _CP + 1
_ES = _L + 1


def _bcast_lane(vec, idx):
    return lax.gather(
        vec, idx[:, None],
        dimension_numbers=lax.GatherDimensionNumbers(
            offset_dims=(), collapsed_slice_dims=(0,), start_index_map=(0,)),
        slice_sizes=(1,),
        mode=lax.GatherScatterMode.PROMISE_IN_BOUNDS)


def _sc_body(tblp_h, p_h, s_h, g_h, b_h, out_h,
             tblp_v, g_v, b_v, pidx_v, sidx_v, e_v,
             out0, out1, semo0, semo1):
    out_v = [out0, out1]
    semo = [semo0, semo1]
    wid = lax.axis_index("s") * _NC + lax.axis_index("c")
    pltpu.sync_copy(tblp_h, tblp_v)
    pltpu.sync_copy(g_h, g_v)
    pltpu.sync_copy(b_h, b_v)
    base0 = wid * _TW
    lane = lax.iota(jnp.int32, _L)
    half = jnp.full((_L,), 0.5, jnp.float32)
    one = jnp.full((_L,), 1, jnp.int32)
    es2 = jnp.full((_L,), 2 * _ES, jnp.int32)
    es1 = jnp.full((_L,), _ES, jnp.int32)
    magic = jnp.full((_L,), 0x5F3759DF, jnp.int32)
    zf = jnp.zeros((_L,), jnp.float32)
    gs = [g_v[pl.ds(cv * _L, _L)] for cv in range(_HV)]
    bs = [b_v[pl.ds(cv * _L, _L)] for cv in range(_HV)]
    ebases = [(cv * _L + lane) * _ES for cv in range(_HV)]

    def unpk(word):
        return plsc.unpack(plsc.bitcast(word, jnp.bfloat16),
                           format=plsc.PackFormat.INTERLEAVED)

    def do_chunk(kk, b):
        base = base0 + kk * _C
        pltpu.sync_copy(p_h.at[pl.ds(base, _C)], pidx_v)
        pltpu.sync_copy(s_h.at[pl.ds(base, _C)], sidx_v)

        @pl.when(kk >= 2)
        def _wait_out():
            pltpu.make_async_copy(
                out_v[b], out_h.at[pl.ds(0, _C * _H)], semo[b]).wait()

        ovb = out_v[b]

        def group_body(g, carry):
            tok0 = g * _L
            pv = pidx_v[pl.ds(tok0, _L)] * _PS
            sv = sidx_v[pl.ds(tok0, _L)] * _PS
            nv = lax.rem(lane + (base + tok0), _N) * _PS

            @plsc.parallel_loop(
                0, _CP, carry=(zf, zf, nv, pv, sv, lane), unroll=2)
            def _p1(cp, cr):
                acc, acc2, ni, pi, si, ei = cr
                n0, n1 = unpk(plsc.load_gather(tblp_v, [ni]))
                p0, p1 = unpk(plsc.load_gather(tblp_v, [pi]))
                s0, s1 = unpk(plsc.load_gather(tblp_v, [si]))
                e0 = n0 + half * (p0 + s0)
                e1 = n1 + half * (p1 + s1)
                plsc.store_scatter(e_v, [ei], e0)
                plsc.store_scatter(e_v, [ei + es1], e1)
                return (acc + (e0 + e1), acc2 + (e0 * e0 + e1 * e1),
                        ni + one, pi + one, si + one, ei + es2)

            acc, acc2 = _p1[0], _p1[1]
            mu = acc * (1.0 / _H)
            var = acc2 * (1.0 / _H) - mu * mu + _EPS
            yi = magic - (plsc.bitcast(var, jnp.int32) >> 1)
            y = plsc.bitcast(yi, jnp.float32)
            for _ in range(3):
                y = y * (1.5 - 0.5 * var * y * y)

            @plsc.parallel_loop(0, _L, unroll=2)
            def _p2(t):
                tsplat = jnp.zeros((_L,), jnp.int32) + t
                mu_sp = _bcast_lane(mu, tsplat)
                inv_sp = _bcast_lane(y, tsplat)
                obase = (tok0 + t) * _H
                for cv in range(_HV):
                    ev = plsc.load_gather(e_v, [ebases[cv] + tsplat])
                    res = (ev - mu_sp) * inv_sp * gs[cv] + bs[cv]
                    ovb[pl.ds(obase + cv * _L, _L)] = res

            return carry

        lax.fori_loop(0, _G, group_body, 0)
        pltpu.async_copy(out_v[b], out_h.at[pl.ds(base * _H, _C * _H)],
                         semo[b])

    def chunk_pair(k2, carry):
        for b in range(2):
            do_chunk(k2 * 2 + b, b)
        return carry

    lax.fori_loop(0, _NCHUNK // 2, chunk_pair, 0)
    for b in range(2):
        pltpu.make_async_copy(
            out_v[b], out_h.at[pl.ds(0, _C * _H)], semo[b]).wait()


def kernel(top_vecs, tok_struct_vec, sent_struct_vec, table, gamma, beta):
    del top_vecs, tok_struct_vec
    p_idx = sent_struct_vec[:, :, 0].reshape(_T).astype(jnp.int32)
    s_idx = sent_struct_vec[:, :, 1].reshape(_T).astype(jnp.int32)
    tb = table.astype(jnp.bfloat16).reshape(_M, _CP, 2)
    packed = jax.lax.bitcast_convert_type(tb, jnp.int32)
    packed = jnp.pad(packed, ((0, 0), (0, 1))).reshape(_M * _PS)
    mesh = plsc.VectorSubcoreMesh(core_axis_name="c", subcore_axis_name="s")
    run = functools.partial(
        pl.kernel,
        mesh=mesh,
        compiler_params=pltpu.CompilerParams(needs_layout_passes=False),
        out_type=jax.ShapeDtypeStruct((_T * _H,), jnp.float32),
        scratch_types=[
            pltpu.VMEM((_M * _PS,), jnp.int32),
            pltpu.VMEM((_H,), jnp.float32),
            pltpu.VMEM((_H,), jnp.float32),
            pltpu.VMEM((_C,), jnp.int32),
            pltpu.VMEM((_C,), jnp.int32),
            pltpu.VMEM((_H * _ES,), jnp.float32),
            pltpu.VMEM((_C * _H,), jnp.float32),
            pltpu.VMEM((_C * _H,), jnp.float32),
            pltpu.SemaphoreType.DMA,
            pltpu.SemaphoreType.DMA,
        ],
    )(_sc_body)
    out = run(packed, p_idx, s_idx, gamma, beta)
    return out.reshape(_B, _N, _H)


# speedup vs baseline: 2.1444x; 1.1247x over previous
import functools

import jax
import jax.numpy as jnp
from jax import lax
from jax.experimental import pallas as pl
from jax.experimental.pallas import tpu as pltpu
from jax.experimental.pallas import tpu_sc as plsc

_B, _N, _H, _M = 1024, 200, 128, 200
_EPS = 1e-12
_T = _B * _N
_NC, _NS, _L = 2, 16, 16
_NW = _NC * _NS
_TW = _T // _NW
_C = 256
_NCHUNK = _TW // _C
_G = _C // _L
_HV = _H // _L
_CP = _H // 2
_PS = _CP + 1
_ES = _L + 1


def _bcast_lane(vec, idx):
    return lax.gather(
        vec, idx[:, None],
        dimension_numbers=lax.GatherDimensionNumbers(
            offset_dims=(), collapsed_slice_dims=(0,), start_index_map=(0,)),
        slice_sizes=(1,),
        mode=lax.GatherScatterMode.PROMISE_IN_BOUNDS)


def _sc_body(tblp_h, p_h, s_h, g_h, b_h, out_h,
             tblp_v, g_v, b_v, pidx_v, sidx_v, e_v,
             out0, out1, semo0, semo1):
    out_v = [out0, out1]
    semo = [semo0, semo1]
    wid = lax.axis_index("s") * _NC + lax.axis_index("c")
    pltpu.sync_copy(tblp_h, tblp_v)
    pltpu.sync_copy(g_h, g_v)
    pltpu.sync_copy(b_h, b_v)
    base0 = wid * _TW
    lane = lax.iota(jnp.int32, _L)
    half = jnp.full((_L,), 0.5, jnp.float32)
    one = jnp.full((_L,), 1, jnp.int32)
    es2 = jnp.full((_L,), 2 * _ES, jnp.int32)
    es1 = jnp.full((_L,), _ES, jnp.int32)
    magic = jnp.full((_L,), 0x5F3759DF, jnp.int32)
    zf = jnp.zeros((_L,), jnp.float32)
    gs = [g_v[pl.ds(cv * _L, _L)] for cv in range(_HV)]
    bs = [b_v[pl.ds(cv * _L, _L)] for cv in range(_HV)]
    ebases = [(cv * _L + lane) * _ES for cv in range(_HV)]

    def unpk(word):
        return plsc.unpack(plsc.bitcast(word, jnp.bfloat16),
                           format=plsc.PackFormat.INTERLEAVED)

    def do_chunk(kk, b):
        base = base0 + kk * _C
        pltpu.sync_copy(p_h.at[pl.ds(base, _C)], pidx_v)
        pltpu.sync_copy(s_h.at[pl.ds(base, _C)], sidx_v)

        @pl.when(kk >= 2)
        def _wait_out():
            pltpu.make_async_copy(
                out_v[b], out_h.at[pl.ds(0, _C * _H)], semo[b]).wait()

        ovb = out_v[b]

        def group_body(g, carry):
            tok0 = g * _L
            pv = pidx_v[pl.ds(tok0, _L)] * _PS
            sv = sidx_v[pl.ds(tok0, _L)] * _PS
            nv = lax.rem(lane + (base + tok0), _N) * _PS

            @plsc.parallel_loop(
                0, _CP, carry=(zf, zf, nv, pv, sv, lane), unroll=4)
            def _p1(cp, cr):
                acc, acc2, ni, pi, si, ei = cr
                n0, n1 = unpk(plsc.load_gather(tblp_v, [ni]))
                p0, p1 = unpk(plsc.load_gather(tblp_v, [pi]))
                s0, s1 = unpk(plsc.load_gather(tblp_v, [si]))
                e0 = n0 + half * (p0 + s0)
                e1 = n1 + half * (p1 + s1)
                plsc.store_scatter(e_v, [ei], e0)
                plsc.store_scatter(e_v, [ei + es1], e1)
                return (acc + (e0 + e1), acc2 + (e0 * e0 + e1 * e1),
                        ni + one, pi + one, si + one, ei + es2)

            acc, acc2 = _p1[0], _p1[1]
            mu = acc * (1.0 / _H)
            var = acc2 * (1.0 / _H) - mu * mu + _EPS
            yi = magic - (plsc.bitcast(var, jnp.int32) >> 1)
            y = plsc.bitcast(yi, jnp.float32)
            for _ in range(3):
                y = y * (1.5 - 0.5 * var * y * y)

            @plsc.parallel_loop(0, _L, unroll=2)
            def _p2(t):
                tsplat = jnp.zeros((_L,), jnp.int32) + t
                mu_sp = _bcast_lane(mu, tsplat)
                inv_sp = _bcast_lane(y, tsplat)
                obase = (tok0 + t) * _H
                for cv in range(_HV):
                    ev = plsc.load_gather(e_v, [ebases[cv] + tsplat])
                    res = (ev - mu_sp) * inv_sp * gs[cv] + bs[cv]
                    ovb[pl.ds(obase + cv * _L, _L)] = res

            return carry

        lax.fori_loop(0, _G, group_body, 0)
        pltpu.async_copy(out_v[b], out_h.at[pl.ds(base * _H, _C * _H)],
                         semo[b])

    def chunk_pair(k2, carry):
        for b in range(2):
            do_chunk(k2 * 2 + b, b)
        return carry

    lax.fori_loop(0, _NCHUNK // 2, chunk_pair, 0)
    for b in range(2):
        pltpu.make_async_copy(
            out_v[b], out_h.at[pl.ds(0, _C * _H)], semo[b]).wait()


def kernel(top_vecs, tok_struct_vec, sent_struct_vec, table, gamma, beta):
    del top_vecs, tok_struct_vec
    p_idx = sent_struct_vec[:, :, 0].reshape(_T).astype(jnp.int32)
    s_idx = sent_struct_vec[:, :, 1].reshape(_T).astype(jnp.int32)
    tb = table.astype(jnp.bfloat16).reshape(_M, _CP, 2)
    packed = jax.lax.bitcast_convert_type(tb, jnp.int32)
    packed = jnp.pad(packed, ((0, 0), (0, 1))).reshape(_M * _PS)
    mesh = plsc.VectorSubcoreMesh(core_axis_name="c", subcore_axis_name="s")
    run = functools.partial(
        pl.kernel,
        mesh=mesh,
        compiler_params=pltpu.CompilerParams(needs_layout_passes=False),
        out_type=jax.ShapeDtypeStruct((_T * _H,), jnp.float32),
        scratch_types=[
            pltpu.VMEM((_M * _PS,), jnp.int32),
            pltpu.VMEM((_H,), jnp.float32),
            pltpu.VMEM((_H,), jnp.float32),
            pltpu.VMEM((_C,), jnp.int32),
            pltpu.VMEM((_C,), jnp.int32),
            pltpu.VMEM((_H * _ES,), jnp.float32),
            pltpu.VMEM((_C * _H,), jnp.float32),
            pltpu.VMEM((_C * _H,), jnp.float32),
            pltpu.SemaphoreType.DMA,
            pltpu.SemaphoreType.DMA,
        ],
    )(_sc_body)
    out = run(packed, p_idx, s_idx, gamma, beta)
    return out.reshape(_B, _N, _H)
